# baseline (speedup 1.0000x reference)
import numpy as np
import jax
import jax.numpy as jnp
from jax.experimental import pallas as pl
from jax.experimental.pallas import tpu as pltpu

_TB = 32
_RPI = 28

_SEGS = (
    (0, 6, 0, 4),
    (4, 10, 4, 4),
    (8, 13, 8, 2),
)


def _build_t1(conv1_w):
    m = np.zeros((8, 28, 4, 2, 16, 3, 3), np.float32)
    for s in range(8):
        for g in range(4):
            di = s - g
            if 0 <= di <= 2:
                for dj in range(3):
                    for par in range(2):
                        for q in range(13):
                            j = 2 * q + par
                            if j <= 25:
                                m[s, j + dj, g, par, q, di, dj] = 1.0
    t = jnp.einsum('swgpqdj,cdj->swgpqc', jnp.asarray(m), conv1_w[:, 0])
    return t.reshape(224, 4096).astype(jnp.bfloat16)


def _build_t2(conv2_w, q_lo, q_hi, j2_lo, nj):
    wq = q_hi - q_lo
    nu = nj // 2
    m = np.zeros((4, wq, 2, 2, nu, 3, 3), np.float32)
    for z in range(4):
        for g2 in range(2):
            di = z - g2
            if 0 <= di <= 2:
                for dj in range(3):
                    for v in range(2):
                        for u in range(nu):
                            q = j2_lo + 2 * u + v + dj
                            if q_lo <= q < q_hi:
                                m[z, q - q_lo, g2, v, u, di, dj] = 1.0
    t = jnp.einsum('zqgvudj,ocdj->zqcgvuo', jnp.asarray(m), conv2_w)
    return t.reshape(4 * wq * 32, 4 * nu * 64).astype(jnp.bfloat16)


def _fused_kernel(x_ref, t1_ref, b1_ref, t2a_ref, t2b_ref, t2c_ref,
                  ba_ref, bb_ref, bc_ref, s_ref, w1_ref, bf1_ref, w2_ref,
                  bf2_ref, out_ref):
    r8 = _TB * (_RPI // 4)
    xp = x_ref[...].astype(jnp.bfloat16)
    xc = jnp.concatenate([xp[0:r8 - 1], xp[1:r8]], axis=1)
    q1 = jnp.dot(xc, t1_ref[...], preferred_element_type=jnp.float32)
    a1 = jnp.maximum(q1 + b1_ref[...], 0.0)
    p0 = jnp.maximum(a1[:, 0:1024], a1[:, 1024:2048])
    p1 = jnp.maximum(a1[:, 2048:3072], a1[:, 3072:4096])
    w0 = jnp.maximum(p0[:, 0:512], p0[:, 512:1024])
    w1 = jnp.maximum(p1[:, 0:512], p1[:, 512:1024])
    h1p = jnp.concatenate([w0, w1], axis=1).astype(jnp.bfloat16)

    pooled = []
    for (q_lo, q_hi, _, nj), t_ref, b_ref in (
            (_SEGS[0], t2a_ref, ba_ref),
            (_SEGS[1], t2b_ref, bb_ref),
            (_SEGS[2], t2c_ref, bc_ref)):
        lo, hi = q_lo * 32, q_hi * 32
        lhs = jnp.concatenate(
            [h1p[0:r8 - 2, lo:hi], h1p[0:r8 - 2, 512 + lo:512 + hi],
             h1p[1:r8 - 1, lo:hi], h1p[1:r8 - 1, 512 + lo:512 + hi]], axis=1)
        q2 = jnp.dot(lhs, t_ref[...], preferred_element_type=jnp.float32)
        a2 = jnp.maximum(q2 + b_ref[...], 0.0)
        half = nj * 64
        g2p = jnp.maximum(a2[:, 0:half], a2[:, half:2 * half])
        pooled.append(jnp.maximum(g2p[:, 0:half // 2], g2p[:, half // 2:half]))
    cp2 = jnp.concatenate(pooled, axis=1)
    h2 = cp2.astype(jnp.bfloat16)

    sel = s_ref[...]
    nsel = r8 - 6
    xfc = jnp.concatenate(
        [jnp.dot(sel, h2[k:nsel + k], preferred_element_type=jnp.float32)
         for k in range(5)], axis=1).astype(jnp.bfloat16)
    f1 = jnp.dot(xfc, w1_ref[...],
                 preferred_element_type=jnp.float32) + bf1_ref[...]
    logits = jnp.dot(f1.astype(jnp.bfloat16), w2_ref[...],
                     preferred_element_type=jnp.float32) + bf2_ref[...]
    mx = jnp.max(logits, axis=-1, keepdims=True)
    e = jnp.exp(logits - mx)
    denom = jnp.sum(e, axis=-1, keepdims=True)
    out_ref[...] = e * pl.reciprocal(denom, approx=True)


def kernel(x, conv1_w, conv1_b, conv2_w, conv2_b, fc1_w, fc1_b, fc2_w, fc2_b):
    b = x.shape[0]
    xp = x.reshape(b * _RPI // 4, 112)

    t1 = _build_t1(conv1_w)
    qmask = np.zeros((2, 16, 1), np.float32)
    qmask[:, :13] = 1.0
    b1row = jnp.tile((jnp.asarray(qmask) * conv1_b).reshape(1, 1024), (1, 4))

    t2s, b2s = [], []
    for q_lo, q_hi, j2_lo, nj in _SEGS:
        t2s.append(_build_t2(conv2_w, q_lo, q_hi, j2_lo, nj))
        b2s.append(jnp.tile(conv2_b, 2 * nj).reshape(1, 2 * nj * 64))

    nrow = _TB * (_RPI // 4)
    sel = np.zeros((_TB, nrow - 6), np.float32)
    for i in range(_TB):
        sel[i, i * (_RPI // 4)] = 1.0
    sel = jnp.asarray(sel).astype(jnp.bfloat16)

    perm = np.zeros(1600, np.int32)
    for pi in range(5):
        for pj in range(5):
            for co in range(64):
                perm[pi * 320 + pj * 64 + co] = co * 25 + pi * 5 + pj
    w1v = jnp.pad(fc1_w[jnp.asarray(perm)], ((0, 0), (0, 64))).astype(jnp.bfloat16)
    bf1 = jnp.pad(fc1_b.reshape(1, 64), ((0, 0), (0, 64)))
    w2p = jnp.pad(fc2_w, ((0, 64), (0, 118))).astype(jnp.bfloat16)
    bf2 = jnp.pad(fc2_b.reshape(1, 10), ((0, 0), (0, 118)),
                  constant_values=-1e9)

    out = pl.pallas_call(
        _fused_kernel,
        out_shape=jax.ShapeDtypeStruct((b, 128), jnp.float32),
        grid=(b // _TB,),
        in_specs=[
            pl.BlockSpec((_TB * (_RPI // 4), 112), lambda i: (i, 0)),
            pl.BlockSpec((224, 4096), lambda i: (0, 0)),
            pl.BlockSpec((1, 4096), lambda i: (0, 0)),
            pl.BlockSpec((768, 512), lambda i: (0, 0)),
            pl.BlockSpec((768, 512), lambda i: (0, 0)),
            pl.BlockSpec((640, 256), lambda i: (0, 0)),
            pl.BlockSpec((1, 512), lambda i: (0, 0)),
            pl.BlockSpec((1, 512), lambda i: (0, 0)),
            pl.BlockSpec((1, 256), lambda i: (0, 0)),
            pl.BlockSpec((_TB, _TB * (_RPI // 4) - 6), lambda i: (0, 0)),
            pl.BlockSpec((1600, 128), lambda i: (0, 0)),
            pl.BlockSpec((1, 128), lambda i: (0, 0)),
            pl.BlockSpec((128, 128), lambda i: (0, 0)),
            pl.BlockSpec((1, 128), lambda i: (0, 0)),
        ],
        out_specs=pl.BlockSpec((_TB, 128), lambda i: (i, 0)),
        compiler_params=pltpu.CompilerParams(
            dimension_semantics=("parallel",)),
    )(xp, t1, b1row, t2s[0], t2s[1], t2s[2], b2s[0], b2s[1], b2s[2],
      sel, w1v, bf1, w2p, bf2)

    return out[:, :10]


# speedup vs baseline: 1.0269x; 1.0269x over previous
import numpy as np
import jax
import jax.numpy as jnp
from jax.experimental import pallas as pl
from jax.experimental.pallas import tpu as pltpu

_TB = 32
_RPI = 28

_SEGS = (
    (0, 6, 0, 4),
    (4, 10, 4, 4),
    (8, 13, 8, 2),
)


def _build_t1(conv1_w):
    m = np.zeros((8, 28, 4, 2, 16, 3, 3), np.float32)
    for s in range(8):
        for g in range(4):
            di = s - g
            if 0 <= di <= 2:
                for dj in range(3):
                    for par in range(2):
                        for q in range(13):
                            j = 2 * q + par
                            if j <= 25:
                                m[s, j + dj, g, par, q, di, dj] = 1.0
    t = jnp.einsum('swgpqdj,cdj->swgpqc', jnp.asarray(m), conv1_w[:, 0])
    return t.reshape(224, 4096).astype(jnp.bfloat16)


def _build_t2(conv2_w, q_lo, q_hi, j2_lo, nj):
    wq = q_hi - q_lo
    nu = nj // 2
    m = np.zeros((4, wq, 2, 2, nu, 3, 3), np.float32)
    for z in range(4):
        for g2 in range(2):
            di = z - g2
            if 0 <= di <= 2:
                for dj in range(3):
                    for v in range(2):
                        for u in range(nu):
                            q = j2_lo + 2 * u + v + dj
                            if q_lo <= q < q_hi:
                                m[z, q - q_lo, g2, v, u, di, dj] = 1.0
    t = jnp.einsum('zqgvudj,ocdj->zqcgvuo', jnp.asarray(m), conv2_w)
    return t.reshape(4 * wq * 32, 4 * nu * 64).astype(jnp.bfloat16)


def _fused_kernel(x_ref, t1_ref, b1_ref, t2a_ref, t2b_ref, t2c_ref,
                  ba_ref, bb_ref, bc_ref, s_ref, w1_ref, bf1_ref, w2_ref,
                  bf2_ref, out_ref):
    r8 = _TB * (_RPI // 4)
    xp = x_ref[...].astype(jnp.bfloat16)
    xc = jnp.concatenate([xp[0:r8 - 1], xp[1:r8]], axis=1)
    q1 = jnp.dot(xc, t1_ref[...], preferred_element_type=jnp.float32)
    a1 = jnp.maximum(q1 + b1_ref[...], 0.0)
    p0 = jnp.maximum(a1[:, 0:1024], a1[:, 1024:2048])
    p1 = jnp.maximum(a1[:, 2048:3072], a1[:, 3072:4096])
    w0 = jnp.maximum(p0[:, 0:512], p0[:, 512:1024])
    w1 = jnp.maximum(p1[:, 0:512], p1[:, 512:1024])
    h1p = jnp.concatenate([w0, w1], axis=1).astype(jnp.bfloat16)

    pooled = []
    for (q_lo, q_hi, _, nj), t_ref, b_ref in (
            (_SEGS[0], t2a_ref, ba_ref),
            (_SEGS[1], t2b_ref, bb_ref),
            (_SEGS[2], t2c_ref, bc_ref)):
        lo, hi = q_lo * 32, q_hi * 32
        lhs = jnp.concatenate(
            [h1p[0:r8 - 2, lo:hi], h1p[0:r8 - 2, 512 + lo:512 + hi],
             h1p[1:r8 - 1, lo:hi], h1p[1:r8 - 1, 512 + lo:512 + hi]], axis=1)
        q2 = jnp.dot(lhs, t_ref[...], preferred_element_type=jnp.float32)
        a2 = jnp.maximum(q2 + b_ref[...], 0.0)
        half = nj * 64
        g2p = jnp.maximum(a2[:, 0:half], a2[:, half:2 * half])
        pooled.append(jnp.maximum(g2p[:, 0:half // 2], g2p[:, half // 2:half]))
    cp2 = jnp.concatenate(pooled, axis=1)
    h2 = cp2.astype(jnp.bfloat16)

    xf = jnp.concatenate([h2[k:r8 - 6 + k] for k in range(5)], axis=1)
    xfc = jnp.dot(s_ref[...], xf,
                  preferred_element_type=jnp.float32).astype(jnp.bfloat16)
    f1 = jnp.dot(xfc, w1_ref[...],
                 preferred_element_type=jnp.float32) + bf1_ref[...]
    logits = jnp.dot(f1.astype(jnp.bfloat16), w2_ref[...],
                     preferred_element_type=jnp.float32) + bf2_ref[...]
    mx = jnp.max(logits, axis=-1, keepdims=True)
    e = jnp.exp(logits - mx)
    denom = jnp.sum(e, axis=-1, keepdims=True)
    sm = e * pl.reciprocal(denom, approx=True)
    out_ref[...] = sm[:, 0:10]


def kernel(x, conv1_w, conv1_b, conv2_w, conv2_b, fc1_w, fc1_b, fc2_w, fc2_b):
    b = x.shape[0]
    xp = x.reshape(b * _RPI // 4, 112)

    t1 = _build_t1(conv1_w)
    qmask = np.zeros((2, 16, 1), np.float32)
    qmask[:, :13] = 1.0
    b1row = jnp.tile((jnp.asarray(qmask) * conv1_b).reshape(1, 1024), (1, 4))

    t2s, b2s = [], []
    for q_lo, q_hi, j2_lo, nj in _SEGS:
        t2s.append(_build_t2(conv2_w, q_lo, q_hi, j2_lo, nj))
        b2s.append(jnp.tile(conv2_b, 2 * nj).reshape(1, 2 * nj * 64))

    nrow = _TB * (_RPI // 4)
    sel = np.zeros((_TB, nrow - 6), np.float32)
    for i in range(_TB):
        sel[i, i * (_RPI // 4)] = 1.0
    sel = jnp.asarray(sel).astype(jnp.bfloat16)

    perm = np.zeros(1600, np.int32)
    for pi in range(5):
        for pj in range(5):
            for co in range(64):
                perm[pi * 320 + pj * 64 + co] = co * 25 + pi * 5 + pj
    w1v = jnp.pad(fc1_w[jnp.asarray(perm)], ((0, 0), (0, 64))).astype(jnp.bfloat16)
    bf1 = jnp.pad(fc1_b.reshape(1, 64), ((0, 0), (0, 64)))
    w2p = jnp.pad(fc2_w, ((0, 64), (0, 118))).astype(jnp.bfloat16)
    bf2 = jnp.pad(fc2_b.reshape(1, 10), ((0, 0), (0, 118)),
                  constant_values=-1e9)

    out = pl.pallas_call(
        _fused_kernel,
        out_shape=jax.ShapeDtypeStruct((b, 10), jnp.float32),
        grid=(b // _TB,),
        in_specs=[
            pl.BlockSpec((_TB * (_RPI // 4), 112), lambda i: (i, 0)),
            pl.BlockSpec((224, 4096), lambda i: (0, 0)),
            pl.BlockSpec((1, 4096), lambda i: (0, 0)),
            pl.BlockSpec((768, 512), lambda i: (0, 0)),
            pl.BlockSpec((768, 512), lambda i: (0, 0)),
            pl.BlockSpec((640, 256), lambda i: (0, 0)),
            pl.BlockSpec((1, 512), lambda i: (0, 0)),
            pl.BlockSpec((1, 512), lambda i: (0, 0)),
            pl.BlockSpec((1, 256), lambda i: (0, 0)),
            pl.BlockSpec((_TB, _TB * (_RPI // 4) - 6), lambda i: (0, 0)),
            pl.BlockSpec((1600, 128), lambda i: (0, 0)),
            pl.BlockSpec((1, 128), lambda i: (0, 0)),
            pl.BlockSpec((128, 128), lambda i: (0, 0)),
            pl.BlockSpec((1, 128), lambda i: (0, 0)),
        ],
        out_specs=pl.BlockSpec((_TB, 10), lambda i: (i, 0)),
        compiler_params=pltpu.CompilerParams(
            dimension_semantics=("parallel",)),
    )(xp, t1, b1row, t2s[0], t2s[1], t2s[2], b2s[0], b2s[1], b2s[2],
      sel, w1v, bf1, w2p, bf2)

    return out
